# baseline (speedup 1.0000x reference)
import functools

import jax
import jax.numpy as jnp
from jax import lax
from jax.experimental import pallas as pl
from jax.experimental.pallas import tpu as pltpu
from jax.experimental.pallas import tpu_sc as plsc

_NC = 2
_NS = 16
_NW = _NC * _NS
_L = 16

_HW = 512 * 512
_K = 4096
_KH = _K // 2
_NIDX = 3 * _KH


def _sc_fsloss_half(disp_flat, idx1d, lam1d):
    mesh = plsc.VectorSubcoreMesh(core_axis_name="c", subcore_axis_name="s")

    @functools.partial(
        pl.kernel,
        out_type=jax.ShapeDtypeStruct((_NW * _L,), jnp.float32),
        mesh=mesh,
        scratch_types=[
            pltpu.VMEM((_NIDX,), jnp.int32),
            pltpu.VMEM((_NIDX,), jnp.float32),
            pltpu.VMEM((_KH,), jnp.float32),
            pltpu.VMEM((_L,), jnp.float32),
            pltpu.SemaphoreType.DMA,
            pltpu.SemaphoreType.DMA,
        ],
    )
    def run(disp_hbm, idx_hbm, lam_hbm, out_hbm, idx_v, vals_v, lam_v,
            part_v, sem, lsem):
        w = lax.axis_index("s") * _NC + lax.axis_index("c")
        p = lax.div(w, 2)
        h = lax.rem(w, 2)
        b = lax.rem(p, 4)

        lam_copy = pltpu.async_copy(
            lam_hbm.at[pl.ds(b * _K + h * _KH, _KH)], lam_v, lsem)
        base = b * 3 * _K + h * _KH
        for j in range(3):
            pltpu.sync_copy(
                idx_hbm.at[pl.ds(base + j * _K, _KH)],
                idx_v.at[pl.ds(j * _KH, _KH)])

        offv = jnp.full((_L,), p * _HW, dtype=jnp.int32)

        nblk = 4
        blk = _NIDX // nblk

        def gather_block(g, carry):
            def add_off(k, carry2):
                s0 = pl.ds(g * blk + k * 2 * _L, _L)
                s1 = pl.ds(g * blk + k * 2 * _L + _L, _L)
                idx_v[s0] = idx_v[s0] + offv
                idx_v[s1] = idx_v[s1] + offv
                return carry2

            lax.fori_loop(0, blk // (2 * _L), add_off, 0)
            sl = pl.ds(g * blk, blk)
            pltpu.async_copy(disp_hbm.at[idx_v.at[sl]], vals_v.at[sl], sem)
            return carry

        lax.fori_loop(0, nblk, gather_block, 0)
        lam_copy.wait()

        def drain(g, carry):
            sl = pl.ds(g * blk, blk)
            pltpu.make_async_copy(
                disp_hbm.at[idx_v.at[sl]], vals_v.at[sl], sem).wait()
            return carry

        lax.fori_loop(0, nblk, drain, 0)

        def body(k, accs):
            a0, a1 = accs
            s0 = pl.ds(k * 2 * _L, _L)
            s1 = pl.ds(k * 2 * _L + _L, _L)
            v1a = vals_v[s0]
            v2a = vals_v[pl.ds(_KH + k * 2 * _L, _L)]
            v3a = vals_v[pl.ds(2 * _KH + k * 2 * _L, _L)]
            lma = lam_v[s0]
            v1b = vals_v[s1]
            v2b = vals_v[pl.ds(_KH + k * 2 * _L + _L, _L)]
            v3b = vals_v[pl.ds(2 * _KH + k * 2 * _L + _L, _L)]
            lmb = lam_v[s1]
            a0 = a0 + jnp.abs(lma * (v2a - v1a) - (v3a - v1a))
            a1 = a1 + jnp.abs(lmb * (v2b - v1b) - (v3b - v1b))
            return (a0, a1)

        zero = jnp.zeros((_L,), jnp.float32)
        a0, a1 = lax.fori_loop(0, _KH // (2 * _L), body, (zero, zero))
        part_v[...] = a0 + a1
        pltpu.sync_copy(part_v, out_hbm.at[pl.ds(w * _L, _L)])

    return run(disp_flat, idx1d, lam1d)


def kernel(disp_preds, keysets, lambda_sets):
    gamma = 0.8
    weight = 1.0
    n_preds = disp_preds.shape[0]
    bs = disp_preds.shape[1]
    k = keysets.shape[-1]
    nh = n_preds // 2

    idx1d = keysets.reshape(-1)
    lam1d = lambda_sets.reshape(-1)

    parts_a = _sc_fsloss_half(disp_preds[:nh].reshape(-1), idx1d, lam1d)
    parts_b = _sc_fsloss_half(disp_preds[nh:].reshape(-1), idx1d, lam1d)
    psum_a = parts_a.reshape(nh, bs * 2 * _L).sum(axis=1)
    psum_b = parts_b.reshape(nh, bs * 2 * _L).sum(axis=1)
    psum = jnp.concatenate([psum_a, psum_b])
    weights = gamma ** jnp.arange(n_preds - 1, -1, -1, dtype=jnp.float32)
    return (psum * weights).sum() / (bs * k) * weight


# speedup vs baseline: 1.4884x; 1.4884x over previous
import functools

import jax
import jax.numpy as jnp
from jax import lax
from jax.experimental import pallas as pl
from jax.experimental.pallas import tpu as pltpu
from jax.experimental.pallas import tpu_sc as plsc

_NC = 2
_NS = 16
_NW = _NC * _NS
_L = 16

_HW = 512 * 512
_K = 4096
_NIDX = 3 * _K
_CH = 128
_NCHUNK = _NIDX // _CH
_NBUF = 8


def _sc_fsloss(disp_flat, idx1d, lam1d):
    mesh = plsc.VectorSubcoreMesh(core_axis_name="c", subcore_axis_name="s")

    @functools.partial(
        pl.kernel,
        out_type=jax.ShapeDtypeStruct((_NW * _L,), jnp.float32),
        mesh=mesh,
        scratch_types=[
            pltpu.VMEM((_NIDX,), jnp.int32),
            pltpu.VMEM((_NIDX,), jnp.float32),
            pltpu.VMEM((_K,), jnp.float32),
            pltpu.VMEM((_L,), jnp.float32),
            pltpu.SemaphoreType.DMA,
            pltpu.SemaphoreType.DMA,
        ],
        compiler_params=pltpu.CompilerParams(skip_device_barrier=True),
    )
    def run(disp_hbm, idx_hbm, lam_hbm, out_hbm, idx_v, vals_v, lam_v,
            part_v, sem, lsem):
        w = lax.axis_index("s") * _NC + lax.axis_index("c")
        b = lax.rem(w, 4)

        lam_copy = pltpu.async_copy(lam_hbm.at[pl.ds(b * _K, _K)], lam_v, lsem)
        pltpu.sync_copy(idx_hbm.at[pl.ds(b * _NIDX, _NIDX)], idx_v)

        offv = jnp.full((_L,), w * _HW, dtype=jnp.int32)

        nblk = 4
        blk = _NIDX // nblk

        def gather_block(g, carry):
            def add_off(k, carry2):
                s0 = pl.ds(g * blk + k * 2 * _L, _L)
                s1 = pl.ds(g * blk + k * 2 * _L + _L, _L)
                idx_v[s0] = idx_v[s0] + offv
                idx_v[s1] = idx_v[s1] + offv
                return carry2

            lax.fori_loop(0, blk // (2 * _L), add_off, 0)
            sl = pl.ds(g * blk, blk)
            pltpu.async_copy(disp_hbm.at[idx_v.at[sl]], vals_v.at[sl], sem)
            return carry

        lax.fori_loop(0, nblk, gather_block, 0)
        lam_copy.wait()

        def drain(g, carry):
            sl = pl.ds(g * blk, blk)
            pltpu.make_async_copy(
                disp_hbm.at[idx_v.at[sl]], vals_v.at[sl], sem).wait()
            return carry

        lax.fori_loop(0, nblk, drain, 0)

        def body(k, accs):
            a0, a1 = accs
            s0 = pl.ds(k * 2 * _L, _L)
            s1 = pl.ds(k * 2 * _L + _L, _L)
            v1a = vals_v[s0]
            v2a = vals_v[pl.ds(_K + k * 2 * _L, _L)]
            v3a = vals_v[pl.ds(2 * _K + k * 2 * _L, _L)]
            lma = lam_v[s0]
            v1b = vals_v[s1]
            v2b = vals_v[pl.ds(_K + k * 2 * _L + _L, _L)]
            v3b = vals_v[pl.ds(2 * _K + k * 2 * _L + _L, _L)]
            lmb = lam_v[s1]
            a0 = a0 + jnp.abs(lma * (v2a - v1a) - (v3a - v1a))
            a1 = a1 + jnp.abs(lmb * (v2b - v1b) - (v3b - v1b))
            return (a0, a1)

        zero = jnp.zeros((_L,), jnp.float32)
        a0, a1 = lax.fori_loop(0, _K // (2 * _L), body, (zero, zero))
        acc = a0 + a1
        part_v[...] = acc
        pltpu.sync_copy(part_v, out_hbm.at[pl.ds(w * _L, _L)])

    return run(disp_flat, idx1d, lam1d)


def kernel(disp_preds, keysets, lambda_sets):
    gamma = 0.8
    weight = 1.0
    n_preds = disp_preds.shape[0]
    bs = disp_preds.shape[1]
    k = keysets.shape[-1]

    disp_flat = disp_preds.reshape(-1)
    idx1d = keysets.reshape(-1)
    lam1d = lambda_sets.reshape(-1)

    parts = _sc_fsloss(disp_flat, idx1d, lam1d)
    psum = parts.reshape(n_preds, bs, _L).sum(axis=(1, 2))
    weights = gamma ** jnp.arange(n_preds - 1, -1, -1, dtype=jnp.float32)
    return (psum * weights).sum() / (bs * k) * weight
